# Initial kernel scaffold; baseline (speedup 1.0000x reference)
import jax
import jax.numpy as jnp
from jax.experimental import pallas as pl


def kernel(x):
    raise NotImplementedError("write your pallas kernel here")


# speedup vs baseline: 2.5710x; 2.5710x over previous
import jax
import jax.numpy as jnp
from jax.experimental import pallas as pl
from jax.experimental.pallas import tpu as pltpu

_BR = 512
_BC = 256


def _body(x_ref, u_ref, o_ref, carry_ref):
    c = pl.program_id(1)

    @pl.when(c == 0)
    def _():
        carry_ref[...] = jnp.zeros_like(carry_ref)

    t = x_ref[...]
    cs = jax.lax.dot(
        t, u_ref[...],
        precision=jax.lax.Precision.HIGHEST,
        preferred_element_type=jnp.float32,
    )
    res = cs + carry_ref[:, 0:1]
    o_ref[...] = res
    carry_ref[...] = jnp.broadcast_to(res[:, -1:], carry_ref.shape)


@jax.jit
def kernel(x):
    R, C = x.shape
    u = jnp.triu(jnp.ones((_BC, _BC), jnp.float32))
    grid = (R // _BR, C // _BC)
    return pl.pallas_call(
        _body,
        grid=grid,
        in_specs=[
            pl.BlockSpec((_BR, _BC), lambda r, c: (r, c)),
            pl.BlockSpec((_BC, _BC), lambda r, c: (0, 0)),
        ],
        out_specs=pl.BlockSpec((_BR, _BC), lambda r, c: (r, c)),
        out_shape=jax.ShapeDtypeStruct((R, C), x.dtype),
        scratch_shapes=[pltpu.VMEM((_BR, 128), jnp.float32)],
        compiler_params=pltpu.CompilerParams(
            dimension_semantics=("parallel", "arbitrary"),
        ),
    )(x, u)
